# baseline (speedup 1.0000x reference)
import jax
import jax.numpy as jnp
from jax import lax
from jax.experimental import pallas as pl
from jax.experimental.pallas import tpu as pltpu
from jax.experimental.pallas import tpu_sc as plsc

BATCH = 16384
EMBED_DIM = 32
NUM_CORES = 2
NUM_SUBCORES = 16
NUM_WORKERS = NUM_CORES * NUM_SUBCORES
ROWS_PER_WORKER = BATCH // NUM_WORKERS
CHUNK = 128
CHUNKS_PER_WORKER = ROWS_PER_WORKER // CHUNK
L = 16
BLOCK_N = 81920


def _proj_tc_kernel(t_ref, w_ref, o_ref):
    o_ref[...] = jnp.sum(t_ref[...] * w_ref[:, 0:1], axis=0, keepdims=True)


def _project(table_t, w_col):
    n = table_t.shape[1]
    grid = (n + BLOCK_N - 1) // BLOCK_N
    out = pl.pallas_call(
        _proj_tc_kernel,
        grid=(grid,),
        in_specs=[
            pl.BlockSpec((EMBED_DIM, BLOCK_N), lambda g: (0, g)),
            pl.BlockSpec((EMBED_DIM, 128), lambda g: (0, 0)),
        ],
        out_specs=pl.BlockSpec((1, BLOCK_N), lambda g: (0, g)),
        out_shape=jax.ShapeDtypeStruct((1, grid * BLOCK_N), jnp.float32),
    )(table_t, w_col)
    return out.reshape(grid * BLOCK_N)


def _gather_sc_kernel(p_u, p_c, u_idx, c_idx, bias16,
                      out, idx_u, idx_c, g_u, g_c, bias_v, out_v, sem):
    wid = lax.axis_index("s") * NUM_CORES + lax.axis_index("c")
    chunk_base = wid * CHUNKS_PER_WORKER

    pltpu.sync_copy(u_idx.at[pl.ds(chunk_base, CHUNKS_PER_WORKER)], idx_u)
    pltpu.sync_copy(c_idx.at[pl.ds(chunk_base, CHUNKS_PER_WORKER)], idx_c)
    pltpu.sync_copy(bias16, bias_v)

    copies = []
    for k in range(CHUNKS_PER_WORKER):
        copies.append(pltpu.async_copy(p_u.at[idx_u.at[k]], g_u.at[k], sem))
        copies.append(pltpu.async_copy(p_c.at[idx_c.at[k]], g_c.at[k], sem))
    for c in copies:
        c.wait()

    bias_vec = bias_v[0:L]
    for k in range(CHUNKS_PER_WORKER):
        for i in range(CHUNK // L):
            s = pl.ds(i * L, L)
            out_v[pl.ds(k * CHUNK + i * L, L)] = g_u[k, s] + g_c[k, s] + bias_vec

    pltpu.sync_copy(out_v, out.at[pl.ds(wid * ROWS_PER_WORKER,
                                        ROWS_PER_WORKER)])


@jax.jit
def _fm(u_idx, c_idx, table_u_t, table_c_t, w_u, w_c, bias16):
    p_u = _project(table_u_t, w_u)
    p_c = _project(table_c_t, w_c)

    mesh = plsc.VectorSubcoreMesh(core_axis_name="c", subcore_axis_name="s")
    run = pl.kernel(
        _gather_sc_kernel,
        out_type=jax.ShapeDtypeStruct((BATCH,), jnp.float32),
        mesh=mesh,
        scratch_types=[
            pltpu.VMEM((CHUNKS_PER_WORKER, CHUNK), jnp.int32),
            pltpu.VMEM((CHUNKS_PER_WORKER, CHUNK), jnp.int32),
            pltpu.VMEM((CHUNKS_PER_WORKER, CHUNK), jnp.float32),
            pltpu.VMEM((CHUNKS_PER_WORKER, CHUNK), jnp.float32),
            pltpu.VMEM((L,), jnp.float32),
            pltpu.VMEM((ROWS_PER_WORKER,), jnp.float32),
            pltpu.SemaphoreType.DMA,
        ],
        compiler_params=pltpu.CompilerParams(needs_layout_passes=False,
                                             use_tc_tiling_on_sc=False),
    )
    return run(p_u, p_c, u_idx, c_idx, bias16)


def kernel(user, course, user_table, course_table, W, b):
    u_idx = user.astype(jnp.int32).reshape(BATCH // CHUNK, CHUNK)
    c_idx = course.astype(jnp.int32).reshape(BATCH // CHUNK, CHUNK)
    w_flat = W.reshape(-1)
    w_u = jnp.broadcast_to(w_flat[:EMBED_DIM, None], (EMBED_DIM, 128))
    w_c = jnp.broadcast_to(w_flat[EMBED_DIM:, None], (EMBED_DIM, 128))
    bias16 = jnp.broadcast_to(b.reshape(-1), (L,))
    out = _fm(u_idx, c_idx, user_table.T, course_table.T, w_u, w_c, bias16)
    return out.reshape(BATCH, 1)


# speedup vs baseline: 1.0233x; 1.0233x over previous
import jax
import jax.numpy as jnp
from jax import lax
from jax.experimental import pallas as pl
from jax.experimental.pallas import tpu as pltpu
from jax.experimental.pallas import tpu_sc as plsc

BATCH = 16384
EMBED_DIM = 32
NUM_CORES = 2
NUM_SUBCORES = 16
NUM_WORKERS = NUM_CORES * NUM_SUBCORES
ROWS_PER_WORKER = BATCH // NUM_WORKERS
CHUNK = 128
CHUNKS_PER_WORKER = ROWS_PER_WORKER // CHUNK
L = 16
BLOCK_N = 73728


def _proj_tc_kernel(t_ref, w_ref, o_ref):
    o_ref[...] = jnp.sum(t_ref[...] * w_ref[:, 0:1], axis=0, keepdims=True)


def _project(table_t, w_col):
    n = table_t.shape[1]
    grid = (n + BLOCK_N - 1) // BLOCK_N
    out = pl.pallas_call(
        _proj_tc_kernel,
        grid=(grid,),
        in_specs=[
            pl.BlockSpec((EMBED_DIM, BLOCK_N), lambda g: (0, g)),
            pl.BlockSpec((EMBED_DIM, 128), lambda g: (0, 0)),
        ],
        out_specs=pl.BlockSpec((1, BLOCK_N), lambda g: (0, g)),
        out_shape=jax.ShapeDtypeStruct((1, grid * BLOCK_N), jnp.float32),
    )(table_t, w_col)
    return out.reshape(grid * BLOCK_N)


def _gather_sc_kernel(p_u, p_c, u_idx, c_idx, bias16,
                      out, idx_u, idx_c, g_u, g_c, bias_v, out_v, sem):
    wid = lax.axis_index("s") * NUM_CORES + lax.axis_index("c")
    chunk_base = wid * CHUNKS_PER_WORKER

    pltpu.sync_copy(u_idx.at[pl.ds(chunk_base, CHUNKS_PER_WORKER)], idx_u)
    pltpu.sync_copy(c_idx.at[pl.ds(chunk_base, CHUNKS_PER_WORKER)], idx_c)
    pltpu.sync_copy(bias16, bias_v)

    copies = []
    for k in range(CHUNKS_PER_WORKER):
        copies.append(pltpu.async_copy(p_u.at[idx_u.at[k]], g_u.at[k], sem))
        copies.append(pltpu.async_copy(p_c.at[idx_c.at[k]], g_c.at[k], sem))
    for c in copies:
        c.wait()

    bias_vec = bias_v[0:L]
    for k in range(CHUNKS_PER_WORKER):
        for i in range(CHUNK // L):
            s = pl.ds(i * L, L)
            out_v[pl.ds(k * CHUNK + i * L, L)] = g_u[k, s] + g_c[k, s] + bias_vec

    pltpu.sync_copy(out_v, out.at[pl.ds(wid * ROWS_PER_WORKER,
                                        ROWS_PER_WORKER)])


@jax.jit
def _fm(u_idx, c_idx, table_u_t, table_c_t, w_u, w_c, bias16):
    p_u = _project(table_u_t, w_u)
    p_c = _project(table_c_t, w_c)

    mesh = plsc.VectorSubcoreMesh(core_axis_name="c", subcore_axis_name="s")
    run = pl.kernel(
        _gather_sc_kernel,
        out_type=jax.ShapeDtypeStruct((BATCH,), jnp.float32),
        mesh=mesh,
        scratch_types=[
            pltpu.VMEM((CHUNKS_PER_WORKER, CHUNK), jnp.int32),
            pltpu.VMEM((CHUNKS_PER_WORKER, CHUNK), jnp.int32),
            pltpu.VMEM((CHUNKS_PER_WORKER, CHUNK), jnp.float32),
            pltpu.VMEM((CHUNKS_PER_WORKER, CHUNK), jnp.float32),
            pltpu.VMEM((L,), jnp.float32),
            pltpu.VMEM((ROWS_PER_WORKER,), jnp.float32),
            pltpu.SemaphoreType.DMA,
        ],
        compiler_params=pltpu.CompilerParams(needs_layout_passes=False,
                                             use_tc_tiling_on_sc=False),
    )
    return run(p_u, p_c, u_idx, c_idx, bias16)


def kernel(user, course, user_table, course_table, W, b):
    u_idx = user.astype(jnp.int32).reshape(BATCH // CHUNK, CHUNK)
    c_idx = course.astype(jnp.int32).reshape(BATCH // CHUNK, CHUNK)
    w_flat = W.reshape(-1)
    w_u = jnp.broadcast_to(w_flat[:EMBED_DIM, None], (EMBED_DIM, 128))
    w_c = jnp.broadcast_to(w_flat[EMBED_DIM:, None], (EMBED_DIM, 128))
    bias16 = jnp.broadcast_to(b.reshape(-1), (L,))
    out = _fm(u_idx, c_idx, user_table.T, course_table.T, w_u, w_c, bias16)
    return out.reshape(BATCH, 1)
